# baseline (speedup 1.0000x reference)
import functools

import jax
import jax.numpy as jnp
from jax import lax
from jax.experimental import pallas as pl
from jax.experimental.pallas import tpu as pltpu
from jax.experimental.pallas import tpu_sc as plsc

N_LABELS = 10
N_NEURONS = 6400
BATCH = 4096

NC = 2
NS = 16
NW = NC * NS
ROWS_PER_W = BATCH // NW
CHUNK = 8
NCHUNKS = ROWS_PER_W // CHUNK
NVREG = N_NEURONS // 16

_mesh = plsc.VectorSubcoreMesh(
    core_axis_name="c", subcore_axis_name="s", num_cores=NC, num_subcores=NS
)


@functools.partial(
    pl.kernel,
    out_type=jax.ShapeDtypeStruct((BATCH,), jnp.int32),
    mesh=_mesh,
    scratch_types=[
        pltpu.VMEM((N_NEURONS,), jnp.int32),
        pltpu.VMEM((CHUNK * N_NEURONS,), jnp.float32),
        pltpu.VMEM((CHUNK * 256,), jnp.float32),
        pltpu.VMEM((256,), jnp.float32),
        pltpu.VMEM((ROWS_PER_W,), jnp.int32),
    ],
    compiler_params=pltpu.CompilerParams(needs_layout_passes=False),
)
def _voting_kernel(fr_hbm, asn_hbm, out_hbm, lbl_v, buf_v, acc_v, tmp_v, out_v):
    wid = lax.axis_index("s") * NC + lax.axis_index("c")
    iota = lax.iota(jnp.int32, 16)
    zeros = jnp.zeros((16,), jnp.float32)
    ones = jnp.ones((16,), jnp.float32)

    pltpu.sync_copy(asn_hbm, lbl_v)

    for l in range(16):
        tmp_v[pl.ds(l * 16, 16)] = zeros
    for r in range(CHUNK * 16):
        acc_v[pl.ds(r * 16, 16)] = zeros

    @pl.loop(0, NVREG)
    def _count(j):
        lbl = lbl_v[pl.ds(j * 16, 16)]
        plsc.addupdate_scatter(tmp_v, [lbl * 16 + iota], ones)

    cnt = zeros
    for l in range(16):
        row = tmp_v[pl.ds(l * 16, 16)]
        plsc.store_scatter(acc_v, [iota * 16 + l], row)
    for l in range(16):
        cnt = cnt + acc_v[pl.ds(l * 16, 16)]
    safe_cnt = jnp.maximum(cnt, 1.0)
    lane_ok = iota < N_LABELS
    cnt_pos = cnt > 0.0

    for r in range(CHUNK * 16):
        acc_v[pl.ds(r * 16, 16)] = zeros

    @pl.loop(0, NCHUNKS)
    def _chunk(c):
        row0 = wid * ROWS_PER_W + c * CHUNK
        pltpu.sync_copy(
            fr_hbm.at[pl.ds(row0 * N_NEURONS, CHUNK * N_NEURONS)], buf_v
        )

        @pl.loop(0, NVREG)
        def _cols(j):
            lbl16 = lbl_v[pl.ds(j * 16, 16)] * 16 + iota
            for r in range(CHUNK):
                v = buf_v[pl.ds(r * N_NEURONS + j * 16, 16)]
                plsc.addupdate_scatter(acc_v, [lbl16 + (256 * r)], v)

        for r in range(CHUNK):
            for l in range(16):
                row = acc_v[pl.ds(r * 256 + l * 16, 16)]
                acc_v[pl.ds(r * 256 + l * 16, 16)] = zeros
                plsc.store_scatter(tmp_v, [iota * 16 + l], row)
            sums = tmp_v[pl.ds(0, 16)]
            for l in range(1, 16):
                sums = sums + tmp_v[pl.ds(l * 16, 16)]
            rates = jnp.where(
                lane_ok, jnp.where(cnt_pos, sums / safe_cnt, 0.0), -1.0
            )
            m = jnp.max(rates)
            winner = plsc.all_reduce_ffs(rates == m)
            pos = c * CHUNK + r
            plsc.store_scatter(
                out_v, [jnp.full((16,), pos, jnp.int32)], winner,
                mask=iota == 0,
            )

    pltpu.sync_copy(out_v, out_hbm.at[pl.ds(wid * ROWS_PER_W, ROWS_PER_W)])


def kernel(firingRate, assignments):
    return _voting_kernel(firingRate.reshape(-1), assignments)


# speedup vs baseline: 1.5632x; 1.5632x over previous
import functools

import jax
import jax.numpy as jnp
from jax import lax
from jax.experimental import pallas as pl
from jax.experimental.pallas import tpu as pltpu
from jax.experimental.pallas import tpu_sc as plsc

N_LABELS = 10
N_NEURONS = 6400
BATCH = 4096

NC = 2
NS = 16
NW = NC * NS
ROWS_PER_W = BATCH // NW
CHUNK = 8
NCHUNKS = ROWS_PER_W // CHUNK
NVREG = N_NEURONS // 16

_mesh = plsc.VectorSubcoreMesh(
    core_axis_name="c", subcore_axis_name="s", num_cores=NC, num_subcores=NS
)


@functools.partial(
    pl.kernel,
    out_type=jax.ShapeDtypeStruct((BATCH,), jnp.int32),
    mesh=_mesh,
    scratch_types=[
        pltpu.VMEM((N_NEURONS,), jnp.int32),
        pltpu.VMEM((2, CHUNK, N_NEURONS), jnp.float32),
        pltpu.VMEM((CHUNK * 256,), jnp.float32),
        pltpu.VMEM((256,), jnp.float32),
        pltpu.VMEM((ROWS_PER_W,), jnp.int32),
        pltpu.SemaphoreType.DMA,
        pltpu.SemaphoreType.DMA,
    ],
    compiler_params=pltpu.CompilerParams(needs_layout_passes=False),
)
def _voting_kernel(
    fr_hbm, asn_hbm, out_hbm, lbl_v, buf_v, acc_v, tmp_v, out_v, sem0, sem1
):
    wid = lax.axis_index("s") * NC + lax.axis_index("c")
    iota = lax.iota(jnp.int32, 16)
    zeros = jnp.zeros((16,), jnp.float32)
    ones = jnp.ones((16,), jnp.float32)
    sems = (sem0, sem1)

    pltpu.sync_copy(asn_hbm, lbl_v)

    for l in range(16):
        tmp_v[pl.ds(l * 16, 16)] = zeros
    for r in range(CHUNK * 16):
        acc_v[pl.ds(r * 16, 16)] = zeros

    @pl.loop(0, NVREG)
    def _count(j):
        idx16 = lbl_v[pl.ds(j * 16, 16)] * 16 + iota
        plsc.addupdate_scatter(tmp_v, [idx16], ones)
        lbl_v[pl.ds(j * 16, 16)] = idx16

    cnt = zeros
    for l in range(16):
        row = tmp_v[pl.ds(l * 16, 16)]
        plsc.store_scatter(acc_v, [iota * 16 + l], row)
    for l in range(16):
        cnt = cnt + acc_v[pl.ds(l * 16, 16)]
    safe_cnt = jnp.maximum(cnt, 1.0)
    lane_ok = iota < N_LABELS
    cnt_pos = cnt > 0.0

    for l in range(16):
        acc_v[pl.ds(l * 16, 16)] = zeros

    def start_dma(c, b):
        row0 = wid * ROWS_PER_W + c * CHUNK
        return pltpu.async_copy(
            fr_hbm.at[pl.ds(row0, CHUNK)], buf_v.at[b], sems[b]
        )

    def wait_dma(b):
        pltpu.make_async_copy(
            fr_hbm.at[pl.ds(0, CHUNK)], buf_v.at[b], sems[b]
        ).wait()

    def process(c, b):
        @pl.loop(0, NVREG, unroll=2)
        def _cols(j):
            idx16 = lbl_v[pl.ds(j * 16, 16)]
            for r in range(CHUNK):
                v = buf_v[b, r, pl.ds(j * 16, 16)]
                plsc.addupdate_scatter(acc_v, [idx16 + (256 * r)], v)

        for r in range(CHUNK):
            for l in range(16):
                row = acc_v[pl.ds(r * 256 + l * 16, 16)]
                acc_v[pl.ds(r * 256 + l * 16, 16)] = zeros
                plsc.store_scatter(tmp_v, [iota * 16 + l], row)
            sums = tmp_v[pl.ds(0, 16)]
            for l in range(1, 16):
                sums = sums + tmp_v[pl.ds(l * 16, 16)]
            rates = jnp.where(
                lane_ok, jnp.where(cnt_pos, sums / safe_cnt, 0.0), -1.0
            )
            m = jnp.max(rates)
            winner = plsc.all_reduce_ffs(rates == m)
            pos = c * CHUNK + r
            plsc.store_scatter(
                out_v, [jnp.full((16,), pos, jnp.int32)], winner,
                mask=iota == 0,
            )

    start_dma(0, 0)

    @pl.loop(0, NCHUNKS, step=2)
    def _chunks(c):
        @pl.when(c + 1 < NCHUNKS)
        def _():
            start_dma(c + 1, 1)

        wait_dma(0)
        process(c, 0)

        @pl.when(c + 2 < NCHUNKS)
        def _():
            start_dma(c + 2, 0)

        @pl.when(c + 1 < NCHUNKS)
        def _():
            wait_dma(1)
            process(c + 1, 1)

    pltpu.sync_copy(out_v, out_hbm.at[pl.ds(wid * ROWS_PER_W, ROWS_PER_W)])


def kernel(firingRate, assignments):
    return _voting_kernel(firingRate, assignments)


# speedup vs baseline: 3.3046x; 2.1140x over previous
import functools

import jax
import jax.numpy as jnp
from jax import lax
from jax.experimental import pallas as pl
from jax.experimental.pallas import tpu as pltpu
from jax.experimental.pallas import tpu_sc as plsc

N_LABELS = 10
N_NEURONS = 6400
BATCH = 4096

NC = 2
NS = 16
NW = NC * NS
ROWS_PER_W = BATCH // NW
CHUNK = 8
NCHUNKS = ROWS_PER_W // CHUNK
NVREG = N_NEURONS // 16

_mesh = plsc.VectorSubcoreMesh(
    core_axis_name="c", subcore_axis_name="s", num_cores=NC, num_subcores=NS
)


@functools.partial(
    pl.kernel,
    out_type=jax.ShapeDtypeStruct((BATCH,), jnp.int32),
    mesh=_mesh,
    scratch_types=[
        pltpu.VMEM((N_NEURONS,), jnp.int32),
        pltpu.VMEM((2, CHUNK, N_NEURONS), jnp.float32),
        pltpu.VMEM((CHUNK * 256,), jnp.float32),
        pltpu.VMEM((256,), jnp.float32),
        pltpu.VMEM((ROWS_PER_W,), jnp.int32),
        pltpu.SemaphoreType.DMA,
        pltpu.SemaphoreType.DMA,
    ],
    compiler_params=pltpu.CompilerParams(needs_layout_passes=False),
)
def _voting_kernel(
    fr_hbm, asn_hbm, out_hbm, lbl_v, buf_v, acc_v, tmp_v, out_v, sem0, sem1
):
    wid = lax.axis_index("s") * NC + lax.axis_index("c")
    iota = lax.iota(jnp.int32, 16)
    zeros = jnp.zeros((16,), jnp.float32)
    ones = jnp.ones((16,), jnp.float32)
    sems = (sem0, sem1)

    pltpu.sync_copy(asn_hbm, lbl_v)

    for l in range(16):
        tmp_v[pl.ds(l * 16, 16)] = zeros
    for r in range(CHUNK * 16):
        acc_v[pl.ds(r * 16, 16)] = zeros

    @pl.loop(0, NVREG)
    def _count(j):
        idx16 = lbl_v[pl.ds(j * 16, 16)] * 16 + iota
        plsc.addupdate_scatter(tmp_v, [idx16], ones)
        lbl_v[pl.ds(j * 16, 16)] = idx16

    cnt = zeros
    for l in range(16):
        row = tmp_v[pl.ds(l * 16, 16)]
        plsc.store_scatter(acc_v, [iota * 16 + l], row)
    for l in range(16):
        cnt = cnt + acc_v[pl.ds(l * 16, 16)]
    safe_cnt = jnp.maximum(cnt, 1.0)
    lane_ok = iota < N_LABELS
    cnt_pos = cnt > 0.0

    for l in range(16):
        acc_v[pl.ds(l * 16, 16)] = zeros

    def start_dma(c, b):
        row0 = wid * ROWS_PER_W + c * CHUNK
        return pltpu.async_copy(
            fr_hbm.at[pl.ds(row0, CHUNK)], buf_v.at[b], sems[b]
        )

    def wait_dma(b):
        pltpu.make_async_copy(
            fr_hbm.at[pl.ds(0, CHUNK)], buf_v.at[b], sems[b]
        ).wait()

    def process(c, b):
        @pl.loop(0, NVREG, unroll=2)
        def _cols(j):
            idx16 = lbl_v[pl.ds(j * 16, 16)]
            vals = [buf_v[b, r, pl.ds(j * 16, 16)] for r in range(CHUNK)]
            idxs = [idx16 + (256 * r) for r in range(CHUNK)]
            for r in range(CHUNK):
                plsc.addupdate_scatter(acc_v, [idxs[r]], vals[r])

        for r in range(CHUNK):
            for l in range(16):
                row = acc_v[pl.ds(r * 256 + l * 16, 16)]
                acc_v[pl.ds(r * 256 + l * 16, 16)] = zeros
                plsc.store_scatter(tmp_v, [iota * 16 + l], row)
            sums = tmp_v[pl.ds(0, 16)]
            for l in range(1, 16):
                sums = sums + tmp_v[pl.ds(l * 16, 16)]
            rates = jnp.where(
                lane_ok, jnp.where(cnt_pos, sums / safe_cnt, 0.0), -1.0
            )
            m = jnp.max(rates)
            winner = plsc.all_reduce_ffs(rates == m)
            pos = c * CHUNK + r
            plsc.store_scatter(
                out_v, [jnp.full((16,), pos, jnp.int32)], winner,
                mask=iota == 0,
            )

    start_dma(0, 0)

    @pl.loop(0, NCHUNKS, step=2)
    def _chunks(c):
        @pl.when(c + 1 < NCHUNKS)
        def _():
            start_dma(c + 1, 1)

        wait_dma(0)
        process(c, 0)

        @pl.when(c + 2 < NCHUNKS)
        def _():
            start_dma(c + 2, 0)

        @pl.when(c + 1 < NCHUNKS)
        def _():
            wait_dma(1)
            process(c + 1, 1)

    pltpu.sync_copy(out_v, out_hbm.at[pl.ds(wid * ROWS_PER_W, ROWS_PER_W)])


def kernel(firingRate, assignments):
    return _voting_kernel(firingRate, assignments)
